# baseline (speedup 1.0000x reference)
import functools

import jax
import jax.numpy as jnp
from jax import lax
from jax.experimental import pallas as pl
from jax.experimental.pallas import tpu as pltpu
from jax.experimental.pallas import tpu_sc as plsc

NUM_TOKENS = 16384 * 50
DIM = 32
NC, NS = 2, 16
NW = NC * NS
PER_W = NUM_TOKENS // NW
GROUP = 1024
NGROUP = PER_W // GROUP
NBUF = 3

_mesh = plsc.VectorSubcoreMesh(core_axis_name="c", subcore_axis_name="s")


@functools.partial(
    pl.kernel,
    out_type=jax.ShapeDtypeStruct((NW * NGROUP, GROUP, DIM), jnp.float32),
    mesh=_mesh,
    scratch_types=[
        pltpu.VMEM((NGROUP, GROUP), jnp.int32),
        pltpu.VMEM((GROUP, DIM), jnp.float32),
        pltpu.VMEM((GROUP, DIM), jnp.float32),
        pltpu.VMEM((GROUP, DIM), jnp.float32),
        pltpu.SemaphoreType.DMA,
        pltpu.SemaphoreType.DMA,
        pltpu.SemaphoreType.DMA,
        pltpu.SemaphoreType.DMA,
        pltpu.SemaphoreType.DMA,
        pltpu.SemaphoreType.DMA,
    ],
    compiler_params=pltpu.CompilerParams(use_tc_tiling_on_sc=False),
)
def _embed_lookup(tok_hbm, table_hbm, out_hbm, idx_v, buf_a, buf_b, buf_c,
                  sem_ga, sem_gb, sem_gc, sem_oa, sem_ob, sem_oc):
    wid = lax.axis_index("s") * NC + lax.axis_index("c")
    gbase = wid * NGROUP
    pltpu.sync_copy(tok_hbm.at[wid], idx_v)

    bufs = (buf_a, buf_b, buf_c)
    gsems = (sem_ga, sem_gb, sem_gc)
    osems = (sem_oa, sem_ob, sem_oc)

    def fire_gather(g):
        b = g % NBUF
        pltpu.async_copy(table_hbm.at[idx_v.at[g]], bufs[b], gsems[b])

    def drain_gather(b):
        pltpu.make_async_copy(out_hbm.at[0], bufs[b], gsems[b]).wait()

    def fire_copy(g, b):
        pltpu.async_copy(bufs[b], out_hbm.at[gbase + g], osems[b])

    def drain_copy(b):
        pltpu.make_async_copy(bufs[b], out_hbm.at[0], osems[b]).wait()

    for g in range(min(NBUF, NGROUP)):
        fire_gather(g)
    for g in range(NGROUP):
        b = g % NBUF
        if g >= NBUF - 1 and g + 1 < NGROUP and g + 1 >= NBUF:
            drain_copy((g + 1) % NBUF)
            fire_gather(g + 1)
        drain_gather(b)
        fire_copy(g, b)
    for b in range(min(NBUF, NGROUP)):
        drain_copy((NGROUP - 1 - b) % NBUF)


def kernel(token_ids, weight):
    tok = token_ids.reshape(NW, NGROUP, GROUP).astype(jnp.int32)
    out = _embed_lookup(tok, weight)
    return out.reshape(token_ids.shape + (DIM,))


# speedup vs baseline: 1.0053x; 1.0053x over previous
import functools

import jax
import jax.numpy as jnp
from jax import lax
from jax.experimental import pallas as pl
from jax.experimental.pallas import tpu as pltpu
from jax.experimental.pallas import tpu_sc as plsc

NUM_TOKENS = 16384 * 50
DIM = 32
NC, NS = 2, 16
NW = NC * NS
PER_W = NUM_TOKENS // NW
GROUP = 1280
NGROUP = PER_W // GROUP

_mesh = plsc.VectorSubcoreMesh(core_axis_name="c", subcore_axis_name="s")


@functools.partial(
    pl.kernel,
    out_type=jax.ShapeDtypeStruct((NW * NGROUP, GROUP, DIM), jnp.float32),
    mesh=_mesh,
    scratch_types=[
        pltpu.VMEM((NGROUP, GROUP), jnp.int32),
        pltpu.VMEM((GROUP, DIM), jnp.float32),
        pltpu.VMEM((GROUP, DIM), jnp.float32),
        pltpu.SemaphoreType.DMA,
        pltpu.SemaphoreType.DMA,
        pltpu.SemaphoreType.DMA,
        pltpu.SemaphoreType.DMA,
    ],
    compiler_params=pltpu.CompilerParams(use_tc_tiling_on_sc=False),
)
def _embed_lookup(tok_hbm, table_hbm, out_hbm, idx_v, buf_a, buf_b,
                  sem_ga, sem_gb, sem_oa, sem_ob):
    wid = lax.axis_index("s") * NC + lax.axis_index("c")
    gbase = wid * NGROUP
    pltpu.sync_copy(tok_hbm.at[wid], idx_v)

    bufs = (buf_a, buf_b)
    gsems = (sem_ga, sem_gb)
    osems = (sem_oa, sem_ob)

    def fire_gather(g, b):
        pltpu.async_copy(table_hbm.at[idx_v.at[g]], bufs[b], gsems[b])

    def drain_gather(b):
        pltpu.make_async_copy(out_hbm.at[0], bufs[b], gsems[b]).wait()

    def fire_copy(g, b):
        pltpu.async_copy(bufs[b], out_hbm.at[gbase + g], osems[b])

    def drain_copy(b):
        pltpu.make_async_copy(bufs[b], out_hbm.at[0], osems[b]).wait()

    fire_gather(0, 0)
    fire_gather(1, 1)
    for g in range(NGROUP):
        b = g % 2
        drain_gather(b)
        fire_copy(g, b)
        if g + 2 < NGROUP:
            drain_copy(b)
            fire_gather(g + 2, b)
    drain_copy(NGROUP % 2)
    drain_copy((NGROUP - 1) % 2)


def kernel(token_ids, weight):
    tok = token_ids.reshape(NW, NGROUP, GROUP).astype(jnp.int32)
    out = _embed_lookup(tok, weight)
    return out.reshape(token_ids.shape + (DIM,))
